# baseline (speedup 1.0000x reference)
import functools

import jax
import jax.numpy as jnp
from jax import lax
from jax.experimental import pallas as pl
from jax.experimental.pallas import tpu as pltpu
from jax.experimental.pallas import tpu_sc as plsc

NUM_EXPERTS = 8
TOP_K = 2
NPAIR = 28
T = 8192
D = 768
DW = D // 2
BB = 256
NPAD = T + NPAIR * BB
NB = NPAD // BB
RB = 1024
NRB = T // RB
NEG = -1e30

NW = 32
TPW = T // NW
CH = 64
NCH = TPW // CH


def _round_f16(x):
    r = jax.lax.bitcast_convert_type(x, jnp.int32)
    r = (r + 0x0FFF + ((r >> 13) & 1)) & ~0x1FFF
    return jax.lax.bitcast_convert_type(r, jnp.float32)


def _router_block(flat_ref, gate_ref, idx_ref, gates_ref, load_ref, dst_ref,
                  bmap_ref, cnt_sc, base_sc, counts_sc, offs_sc):
    i = pl.program_id(0)

    @pl.when(i < NRB)
    def _phase_a():
        x = flat_ref[...]
        g = gate_ref[...]
        logits = jnp.dot(x, g, preferred_element_type=jnp.float32)
        col = jax.lax.broadcasted_iota(jnp.int32, logits.shape, 1)
        logits = jnp.where(col < NUM_EXPERTS, logits, NEG)

        v1 = jnp.max(logits, axis=1, keepdims=True)
        i1 = jnp.min(jnp.where(logits == v1, col, 128), axis=1, keepdims=True)
        l2 = jnp.where(col == i1, NEG, logits)
        v2 = jnp.max(l2, axis=1, keepdims=True)
        i2 = jnp.min(jnp.where(l2 == v2, col, 128), axis=1, keepdims=True)

        e2v = jnp.exp(v2 - v1)
        g1 = _round_f16(1.0 / (1.0 + e2v))
        g2 = _round_f16(e2v / (1.0 + e2v))

        gates_blk = jnp.where(col == i1, g1, 0.0) + jnp.where(col == i2, g2, 0.0)
        gates_ref[...] = gates_blk
        idx_ref[...] = jnp.where(col == 0, i1, jnp.where(col == 1, i2, 0))

        @pl.when(i == 0)
        def _():
            load_ref[...] = jnp.zeros_like(load_ref)
            counts_sc[...] = jnp.zeros_like(counts_sc)

        load_ref[...] += jnp.sum((gates_blk > 0).astype(jnp.int32), axis=0,
                                 keepdims=True)

        pa = jnp.minimum(i1, i2)
        pb = jnp.maximum(i1, i2)
        pair = (pa * (15 - pa)) // 2 + (pb - pa - 1)
        cnt = (col == pair).astype(jnp.float32)

        r_iota = jax.lax.broadcasted_iota(jnp.int32, (RB, RB), 0)
        c_iota = jax.lax.broadcasted_iota(jnp.int32, (RB, RB), 1)
        tri = (c_iota < r_iota).astype(jnp.float32)
        excl = jnp.dot(tri, cnt, preferred_element_type=jnp.float32)
        base = excl + counts_sc[...]

        cnt_sc[pl.ds(i * RB, RB), :] = cnt
        base_sc[pl.ds(i * RB, RB), :] = base
        counts_sc[...] += jnp.sum(cnt, axis=0, keepdims=True)

    @pl.when(i == NRB)
    def _phase_b_setup():
        c = counts_sc[...]
        r = jnp.floor((c + (BB - 1)) / BB) * BB
        k_iota = jax.lax.broadcasted_iota(jnp.int32, (128, 128), 0)
        p_iota = jax.lax.broadcasted_iota(jnp.int32, (128, 128), 1)
        tri = (k_iota < p_iota).astype(jnp.float32)
        excl_off = jnp.dot(r, tri, preferred_element_type=jnp.float32)
        offs_sc[...] = excl_off

        incl = excl_off + r
        incl_mat = jnp.broadcast_to(incl, (128, 128))
        bstart = (jax.lax.broadcasted_iota(jnp.int32, (128, 128), 0)
                  .astype(jnp.float32) * BB)
        used = ((incl_mat <= bstart) & (p_iota < NPAIR)).astype(jnp.float32)
        bmap = jnp.sum(used, axis=1, keepdims=True)
        bmap = jnp.minimum(bmap, NPAIR - 1).astype(jnp.int32)
        bmap_ref[...] = jnp.broadcast_to(bmap, (128, 128))

    @pl.when(i >= NRB)
    def _phase_b():
        s = i - NRB
        cnt = cnt_sc[pl.ds(s * RB, RB), :]
        base = base_sc[pl.ds(s * RB, RB), :]
        offs = offs_sc[...]
        dst = jnp.sum(cnt * (offs + base), axis=1, keepdims=True)
        dst_ref[...] = jnp.broadcast_to(dst.astype(jnp.int32), (RB, 128))


def _router(flat, gate_pad):
    return pl.pallas_call(
        _router_block,
        grid=(2 * NRB,),
        in_specs=[
            pl.BlockSpec((RB, D), lambda i: (jnp.minimum(i, NRB - 1), 0)),
            pl.BlockSpec((D, 128), lambda i: (0, 0)),
        ],
        out_specs=[
            pl.BlockSpec((RB, 128), lambda i: (jnp.minimum(i, NRB - 1), 0)),
            pl.BlockSpec((RB, 128), lambda i: (jnp.minimum(i, NRB - 1), 0)),
            pl.BlockSpec((1, 128), lambda i: (0, 0)),
            pl.BlockSpec((RB, 128), lambda i: (jnp.maximum(i - NRB, 0), 0)),
            pl.BlockSpec((128, 128), lambda i: (0, 0)),
        ],
        out_shape=[
            jax.ShapeDtypeStruct((T, 128), jnp.int32),
            jax.ShapeDtypeStruct((T, 128), jnp.float32),
            jax.ShapeDtypeStruct((1, 128), jnp.int32),
            jax.ShapeDtypeStruct((T, 128), jnp.int32),
            jax.ShapeDtypeStruct((128, 128), jnp.int32),
        ],
        scratch_shapes=[
            pltpu.VMEM((T, 128), jnp.float32),
            pltpu.VMEM((T, 128), jnp.float32),
            pltpu.VMEM((1, 128), jnp.float32),
            pltpu.VMEM((1, 128), jnp.float32),
        ],
    )(flat, gate_pad)


@functools.cache
def _sc_kernels():
    mesh = plsc.VectorSubcoreMesh(core_axis_name="c", subcore_axis_name="s")

    @functools.partial(
        pl.kernel,
        out_type=jax.ShapeDtypeStruct((NPAD, DW), jnp.float32),
        mesh=mesh,
        scratch_types=[
            pltpu.VMEM((CH,), jnp.int32),
            pltpu.VMEM((CH, DW), jnp.float32),
            pltpu.SemaphoreType.DMA,
        ],
    )
    def dispatch_sc(flat_view, dst_vec, xg_view, idx_v, rows_v, sem):
        wid = lax.axis_index("s") * 2 + lax.axis_index("c")
        base = wid * TPW
        for c in range(NCH):
            off = base + c * CH
            pltpu.sync_copy(dst_vec.at[pl.ds(off, CH)], idx_v)
            pltpu.sync_copy(flat_view.at[pl.ds(off, CH)], rows_v)
            pltpu.async_copy(rows_v, xg_view.at[idx_v], sem).wait()

    @functools.partial(
        pl.kernel,
        out_type=jax.ShapeDtypeStruct((T, D), jnp.float32),
        mesh=mesh,
        scratch_types=[
            pltpu.VMEM((CH,), jnp.int32),
            pltpu.VMEM((CH, D), jnp.float32),
            pltpu.SemaphoreType.DMA,
        ],
    )
    def combine_sc(y, dst_vec, res, idx_v, rows_v, sem):
        wid = lax.axis_index("s") * 2 + lax.axis_index("c")
        base = wid * TPW
        for c in range(NCH):
            off = base + c * CH
            pltpu.sync_copy(dst_vec.at[pl.ds(off, CH)], idx_v)
            pltpu.async_copy(y.at[idx_v], rows_v, sem).wait()
            pltpu.sync_copy(rows_v, res.at[pl.ds(off, CH)])

    return dispatch_sc, combine_sc


def _mm_block(bmap_ref, x_ref, w_ref, y_ref):
    p = bmap_ref[pl.program_id(0)]
    e1 = jnp.int32(0)
    for th in (7, 13, 18, 22, 25, 27):
        e1 = e1 + (p >= th).astype(jnp.int32)
    e2 = p - (e1 * (15 - e1)) // 2 + e1 + 1
    ws = (w_ref[pl.ds(e1, 1)] + w_ref[pl.ds(e2, 1)])[0]
    y_ref[...] = jnp.dot(x_ref[...], ws, preferred_element_type=jnp.float32)


def _matmul(bmap, xg_bf, w_bf):
    return pl.pallas_call(
        _mm_block,
        grid_spec=pltpu.PrefetchScalarGridSpec(
            num_scalar_prefetch=1,
            grid=(NB,),
            in_specs=[
                pl.BlockSpec((BB, D), lambda b, m: (b, 0)),
                pl.BlockSpec((NUM_EXPERTS, D, D), lambda b, m: (0, 0, 0)),
            ],
            out_specs=pl.BlockSpec((BB, D), lambda b, m: (b, 0)),
        ),
        out_shape=jax.ShapeDtypeStruct((NPAD, D), jnp.float32),
    )(bmap, xg_bf, w_bf)


def kernel(inputs, clean_gate, noise_gate, expert_W, patch_h, patch_w):
    b, s, dim = inputs.shape
    flat = inputs.reshape(-1, dim)

    gate_pad = jnp.zeros((dim, 128), jnp.float32).at[:, :NUM_EXPERTS].set(clean_gate)
    idx_o, gates_o, load_o, dst_o, bmap_o = _router(flat, gate_pad)

    dst_vec = dst_o[:, 0]
    bmap = bmap_o[:NB, 0]

    flat_bf = flat.astype(jnp.bfloat16)
    flat_view = jax.lax.bitcast_convert_type(flat_bf.reshape(T, DW, 2),
                                             jnp.float32)
    dispatch_sc, combine_sc = _sc_kernels()
    xg_view = dispatch_sc(flat_view, dst_vec)
    xg_bf = jax.lax.bitcast_convert_type(xg_view, jnp.bfloat16).reshape(NPAD, D)

    y = _matmul(bmap, xg_bf, expert_W.astype(jnp.bfloat16))
    res = combine_sc(y, dst_vec)

    return (res.reshape(b, s, D), idx_o[:, :TOP_K], gates_o[:, :NUM_EXPERTS],
            load_o[0, :NUM_EXPERTS])


# speedup vs baseline: 3.1049x; 3.1049x over previous
import functools

import jax
import jax.numpy as jnp
from jax import lax
from jax.experimental import pallas as pl
from jax.experimental.pallas import tpu as pltpu
from jax.experimental.pallas import tpu_sc as plsc

NUM_EXPERTS = 8
TOP_K = 2
NPAIR = 28
T = 8192
D = 768
DW = D // 2
BB = 256
NPAD = T + NPAIR * BB
NB = NPAD // BB
RB = 1024
NRB = T // RB
NEG = -1e30

NW = 32
TPW = T // NW
CH = 64
NCH = TPW // CH


def _round_f16(x):
    r = jax.lax.bitcast_convert_type(x, jnp.int32)
    r = (r + 0x0FFF + ((r >> 13) & 1)) & ~0x1FFF
    return jax.lax.bitcast_convert_type(r, jnp.float32)


def _router_block(flat_ref, gate_ref, idx_ref, gates_ref, load_ref, dst_ref,
                  bmap_ref, cnt_sc, base_sc, counts_sc, offs_sc):
    i = pl.program_id(0)

    @pl.when(i < NRB)
    def _phase_a():
        x = flat_ref[...]
        g = gate_ref[...]
        logits = jnp.dot(x, g, preferred_element_type=jnp.float32)
        col = jax.lax.broadcasted_iota(jnp.int32, logits.shape, 1)
        logits = jnp.where(col < NUM_EXPERTS, logits, NEG)

        v1 = jnp.max(logits, axis=1, keepdims=True)
        i1 = jnp.min(jnp.where(logits == v1, col, 128), axis=1, keepdims=True)
        l2 = jnp.where(col == i1, NEG, logits)
        v2 = jnp.max(l2, axis=1, keepdims=True)
        i2 = jnp.min(jnp.where(l2 == v2, col, 128), axis=1, keepdims=True)

        e2v = jnp.exp(v2 - v1)
        g1 = _round_f16(1.0 / (1.0 + e2v))
        g2 = _round_f16(e2v / (1.0 + e2v))

        gates_blk = jnp.where(col == i1, g1, 0.0) + jnp.where(col == i2, g2, 0.0)
        gates_ref[...] = gates_blk
        idx_ref[...] = jnp.where(col == 0, i1, jnp.where(col == 1, i2, 0))

        @pl.when(i == 0)
        def _():
            load_ref[...] = jnp.zeros_like(load_ref)
            counts_sc[...] = jnp.zeros_like(counts_sc)

        load_ref[...] += jnp.sum((gates_blk > 0).astype(jnp.int32), axis=0,
                                 keepdims=True)

        pa = jnp.minimum(i1, i2)
        pb = jnp.maximum(i1, i2)
        pair = (pa * (15 - pa)) // 2 + (pb - pa - 1)
        cnt = (col == pair).astype(jnp.float32)

        r_iota = jax.lax.broadcasted_iota(jnp.int32, (RB, RB), 0)
        c_iota = jax.lax.broadcasted_iota(jnp.int32, (RB, RB), 1)
        tri = (c_iota < r_iota).astype(jnp.float32)
        excl = jnp.dot(tri, cnt, preferred_element_type=jnp.float32)
        base = excl + counts_sc[...]

        cnt_sc[pl.ds(i * RB, RB), :] = cnt
        base_sc[pl.ds(i * RB, RB), :] = base
        counts_sc[...] += jnp.sum(cnt, axis=0, keepdims=True)

    @pl.when(i == NRB)
    def _phase_b_setup():
        c = counts_sc[...]
        r = jnp.floor((c + (BB - 1)) / BB) * BB
        k_iota = jax.lax.broadcasted_iota(jnp.int32, (128, 128), 0)
        p_iota = jax.lax.broadcasted_iota(jnp.int32, (128, 128), 1)
        tri = (k_iota < p_iota).astype(jnp.float32)
        excl_off = jnp.dot(r, tri, preferred_element_type=jnp.float32)
        offs_sc[...] = excl_off

        incl = excl_off + r
        incl_mat = jnp.broadcast_to(incl, (128, 128))
        bstart = (jax.lax.broadcasted_iota(jnp.int32, (128, 128), 0)
                  .astype(jnp.float32) * BB)
        used = ((incl_mat <= bstart) & (p_iota < NPAIR)).astype(jnp.float32)
        bmap = jnp.sum(used, axis=1, keepdims=True)
        bmap = jnp.minimum(bmap, NPAIR - 1).astype(jnp.int32)
        bmap_ref[...] = jnp.broadcast_to(bmap, (128, 128))

    @pl.when(i >= NRB)
    def _phase_b():
        s = i - NRB
        cnt = cnt_sc[pl.ds(s * RB, RB), :]
        base = base_sc[pl.ds(s * RB, RB), :]
        offs = offs_sc[...]
        dst = jnp.sum(cnt * (offs + base), axis=1)
        dst_ref[...] = jnp.reshape(dst.astype(jnp.int32), (RB // 128, 128))


def _router(flat, gate_pad):
    return pl.pallas_call(
        _router_block,
        grid=(2 * NRB,),
        in_specs=[
            pl.BlockSpec((RB, D), lambda i: (jnp.minimum(i, NRB - 1), 0)),
            pl.BlockSpec((D, 128), lambda i: (0, 0)),
        ],
        out_specs=[
            pl.BlockSpec((RB, 128), lambda i: (jnp.minimum(i, NRB - 1), 0)),
            pl.BlockSpec((RB, 128), lambda i: (jnp.minimum(i, NRB - 1), 0)),
            pl.BlockSpec((1, 128), lambda i: (0, 0)),
            pl.BlockSpec((RB // 128, 128), lambda i: (jnp.maximum(i - NRB, 0), 0)),
            pl.BlockSpec((128, 128), lambda i: (0, 0)),
        ],
        out_shape=[
            jax.ShapeDtypeStruct((T, 128), jnp.int32),
            jax.ShapeDtypeStruct((T, 128), jnp.float32),
            jax.ShapeDtypeStruct((1, 128), jnp.int32),
            jax.ShapeDtypeStruct((T // 128, 128), jnp.int32),
            jax.ShapeDtypeStruct((128, 128), jnp.int32),
        ],
        scratch_shapes=[
            pltpu.VMEM((T, 128), jnp.float32),
            pltpu.VMEM((T, 128), jnp.float32),
            pltpu.VMEM((1, 128), jnp.float32),
            pltpu.VMEM((1, 128), jnp.float32),
        ],
    )(flat, gate_pad)


@functools.cache
def _sc_kernels():
    mesh = plsc.VectorSubcoreMesh(core_axis_name="c", subcore_axis_name="s")

    @functools.partial(
        pl.kernel,
        out_type=jax.ShapeDtypeStruct((NPAD, D), jnp.float32),
        mesh=mesh,
        scratch_types=[
            pltpu.VMEM((CH,), jnp.int32),
            pltpu.VMEM((CH, D), jnp.float32),
            pltpu.SemaphoreType.DMA,
        ],
    )
    def dispatch_sc(flat, dst_f, xg, idx_v, rows_v, sem):
        wid = lax.axis_index("s") * 2 + lax.axis_index("c")
        base = wid * TPW
        for c in range(NCH):
            off = base + c * CH
            pltpu.sync_copy(dst_f.at[pl.ds(off, CH)], idx_v)
            pltpu.sync_copy(flat.at[pl.ds(off, CH)], rows_v)
            pltpu.async_copy(rows_v, xg.at[idx_v], sem).wait()

    @functools.partial(
        pl.kernel,
        out_type=jax.ShapeDtypeStruct((T, D), jnp.float32),
        mesh=mesh,
        scratch_types=[
            pltpu.VMEM((CH,), jnp.int32),
            pltpu.VMEM((CH, D), jnp.float32),
            pltpu.SemaphoreType.DMA,
        ],
    )
    def combine_sc(y, dst_f, res, idx_v, rows_v, sem):
        wid = lax.axis_index("s") * 2 + lax.axis_index("c")
        base = wid * TPW
        for c in range(NCH):
            off = base + c * CH
            pltpu.sync_copy(dst_f.at[pl.ds(off, CH)], idx_v)
            pltpu.async_copy(y.at[idx_v], rows_v, sem).wait()
            pltpu.sync_copy(rows_v, res.at[pl.ds(off, CH)])

    return dispatch_sc, combine_sc


def _mm_block(bmap_ref, x_ref, w_ref, y_ref):
    p = bmap_ref[pl.program_id(0)]
    e1 = jnp.int32(0)
    for th in (7, 13, 18, 22, 25, 27):
        e1 = e1 + (p >= th).astype(jnp.int32)
    e2 = p - (e1 * (15 - e1)) // 2 + e1 + 1
    ws = (w_ref[pl.ds(e1, 1)] + w_ref[pl.ds(e2, 1)])[0].astype(jnp.bfloat16)
    xb = x_ref[...].astype(jnp.bfloat16)
    y_ref[...] = jnp.dot(xb, ws, preferred_element_type=jnp.float32)


def _matmul(bmap, xg_bf, w_bf):
    return pl.pallas_call(
        _mm_block,
        grid_spec=pltpu.PrefetchScalarGridSpec(
            num_scalar_prefetch=1,
            grid=(NB,),
            in_specs=[
                pl.BlockSpec((BB, D), lambda b, m: (b, 0)),
                pl.BlockSpec((NUM_EXPERTS, D, D), lambda b, m: (0, 0, 0)),
            ],
            out_specs=pl.BlockSpec((BB, D), lambda b, m: (b, 0)),
        ),
        out_shape=jax.ShapeDtypeStruct((NPAD, D), jnp.float32),
    )(bmap, xg_bf, w_bf)


def kernel(inputs, clean_gate, noise_gate, expert_W, patch_h, patch_w):
    b, s, dim = inputs.shape
    flat = inputs.reshape(-1, dim)

    gate_pad = jnp.zeros((dim, 128), jnp.float32).at[:, :NUM_EXPERTS].set(clean_gate)
    idx_o, gates_o, load_o, dst_o, bmap_o = _router(flat, gate_pad)

    bmap = bmap_o[:NB, 0]

    dst_f = dst_o.reshape(-1)
    dispatch_sc, combine_sc = _sc_kernels()
    xg = dispatch_sc(flat, dst_f)
    y = _matmul(bmap, xg, expert_W)
    res = combine_sc(y, dst_f)

    return (res.reshape(b, s, D), idx_o[:, :TOP_K], gates_o[:, :NUM_EXPERTS],
            load_o[0, :NUM_EXPERTS])
